# baseline (speedup 1.0000x reference)
import functools

import jax
import jax.numpy as jnp
from jax import lax
from jax.experimental import pallas as pl
from jax.experimental.pallas import tpu as pltpu
from jax.experimental.pallas import tpu_sc as plsc

CHUNK = 16
NBUF = 4
LEAD = 2


@functools.lru_cache(maxsize=None)
def _make_sc_gather(N, V, D, nc, ns):
    nw = nc * ns
    n_per_w = N // nw
    n_chunks = n_per_w // CHUNK
    n_mid = (n_chunks - 2 * NBUF) // NBUF
    tail_start = NBUF * (1 + n_mid)
    assert n_chunks >= 3 * NBUF and 1 <= LEAD < NBUF
    mesh = plsc.VectorSubcoreMesh(core_axis_name="c", subcore_axis_name="s")

    @functools.partial(
        pl.kernel,
        mesh=mesh,
        out_type=jax.ShapeDtypeStruct((N, D), jnp.float32),
        scratch_types=[
            pltpu.VMEM((n_per_w,), jnp.int32),
            pltpu.VMEM((NBUF, CHUNK, D), jnp.float32),
        ]
        + [pltpu.SemaphoreType.DMA] * (2 * NBUF),
    )
    def gather_kernel(idx_hbm, pe_hbm, out_hbm, idx_v, rows_v, *sems):
        gsem, ssem = sems[:NBUF], sems[NBUF:]
        wid = lax.axis_index("s") * nc + lax.axis_index("c")
        base = wid * n_per_w
        pltpu.sync_copy(idx_hbm.at[pl.ds(base, n_per_w)], idx_v)

        def gather(c, b):
            return pltpu.make_async_copy(
                pe_hbm.at[idx_v.at[pl.ds(c * CHUNK, CHUNK)]],
                rows_v.at[b],
                gsem[b],
            )

        def store(c, b):
            return pltpu.make_async_copy(
                rows_v.at[b],
                out_hbm.at[pl.ds(base + c * CHUNK, CHUNK)],
                ssem[b],
            )

        def emit(c, cc, b):
            gather(cc, b).wait()
            store(cc, b).start()
            if c - (NBUF - LEAD) >= 0:
                store(cc - (NBUF - LEAD), (c - (NBUF - LEAD)) % NBUF).wait()
            if c + LEAD < n_chunks:
                gather(cc + LEAD, (c + LEAD) % NBUF).start()

        for b in range(LEAD):
            gather(b, b).start()
        for c in range(NBUF):
            emit(c, c, c % NBUF)

        def body(i, carry):
            c0 = (i + 1) * NBUF
            for b in range(NBUF):
                emit(NBUF + b, c0 + b, b)
            return carry

        lax.fori_loop(0, n_mid, body, 0)

        for c in range(tail_start, n_chunks):
            emit(c, c, c % NBUF)
        for c in range(n_chunks - (NBUF - LEAD), n_chunks):
            store(c, c % NBUF).wait()

    return gather_kernel


def kernel(position_ids, pe):
    B, T = position_ids.shape
    V, D = pe.shape
    N = B * T
    info = plsc.get_sparse_core_info()
    idx = position_ids.reshape(N).astype(jnp.int32)
    out = _make_sc_gather(N, V, D, info.num_cores, info.num_subcores)(idx, pe)
    return out.reshape(B, T, D)


# speedup vs baseline: 1.0037x; 1.0037x over previous
import functools

import jax
import jax.numpy as jnp
from jax import lax
from jax.experimental import pallas as pl
from jax.experimental.pallas import tpu as pltpu
from jax.experimental.pallas import tpu_sc as plsc

CHUNK = 16
NBUF = 4
LEAD = 2


@functools.lru_cache(maxsize=None)
def _make_sc_gather(N, V, D, nc, ns):
    nw = nc * ns
    n_per_w = N // nw
    n_chunks = n_per_w // CHUNK
    n_mid = (n_chunks - 2 * NBUF) // NBUF
    tail_start = NBUF * (1 + n_mid)
    assert n_chunks >= 3 * NBUF and 1 <= LEAD < NBUF
    mesh = plsc.VectorSubcoreMesh(core_axis_name="c", subcore_axis_name="s")

    @functools.partial(
        pl.kernel,
        mesh=mesh,
        out_type=jax.ShapeDtypeStruct((N, D), jnp.float32),
        scratch_types=[
            pltpu.VMEM((n_per_w,), jnp.int32),
            pltpu.VMEM((NBUF, CHUNK, D), jnp.float32),
        ]
        + [pltpu.SemaphoreType.DMA] * (2 * NBUF),
    )
    def gather_kernel(idx_hbm, pe_hbm, out_hbm, idx_v, rows_v, *sems):
        gsem, ssem = sems[:NBUF], sems[NBUF:]
        wid = lax.axis_index("s") * nc + lax.axis_index("c")
        base = wid * n_per_w
        pltpu.sync_copy(idx_hbm.at[pl.ds(base, n_per_w)], idx_v)

        def gather(c, b):
            return pltpu.make_async_copy(
                pe_hbm.at[idx_v.at[pl.ds(c * CHUNK, CHUNK)]],
                rows_v.at[b],
                gsem[b],
            )

        def store(c, b):
            return pltpu.make_async_copy(
                rows_v.at[b],
                out_hbm.at[pl.ds(base + c * CHUNK, CHUNK)],
                ssem[b],
            )

        def emit(c, cc, b):
            gather(cc, b).wait()
            if c - (NBUF - LEAD) >= 0:
                store(cc - (NBUF - LEAD), (c - (NBUF - LEAD)) % NBUF).wait()
            if c + LEAD < n_chunks:
                gather(cc + LEAD, (c + LEAD) % NBUF).start()
            store(cc, b).start()

        for b in range(LEAD):
            gather(b, b).start()
        for c in range(NBUF):
            emit(c, c, c % NBUF)

        def body(i, carry):
            c0 = (i + 1) * NBUF
            for b in range(NBUF):
                emit(NBUF + b, c0 + b, b)
            return carry

        lax.fori_loop(0, n_mid, body, 0)

        for c in range(tail_start, n_chunks):
            emit(c, c, c % NBUF)
        for c in range(n_chunks - (NBUF - LEAD), n_chunks):
            store(c, c % NBUF).wait()

    return gather_kernel


def kernel(position_ids, pe):
    B, T = position_ids.shape
    V, D = pe.shape
    N = B * T
    info = plsc.get_sparse_core_info()
    idx = position_ids.reshape(N).astype(jnp.int32)
    out = _make_sc_gather(N, V, D, info.num_cores, info.num_subcores)(idx, pe)
    return out.reshape(B, T, D)
